# Initial kernel scaffold; baseline (speedup 1.0000x reference)
import jax
import jax.numpy as jnp
from jax.experimental import pallas as pl


def kernel(x, fp3_values):
    raise NotImplementedError("write your pallas kernel here")


# speedup vs baseline: 4742.7198x; 4742.7198x over previous
import functools

import jax
import jax.numpy as jnp
from jax import lax
from jax.experimental import pallas as pl
from jax.experimental.pallas import tpu as pltpu
from jax.experimental.pallas import tpu_sc as plsc

_NC = 2
_NS = 16
_NW = _NC * _NS
_L = 16
_C = 16384
_U = 8


def _make_sc_quantize(n):
    per_w = n // _NW
    nchunks = per_w // _C
    assert per_w % _C == 0 and nchunks % 2 == 0

    mesh = plsc.VectorSubcoreMesh(core_axis_name="c", subcore_axis_name="s")

    @functools.partial(
        pl.kernel,
        out_type=jax.ShapeDtypeStruct((n,), jnp.float32),
        mesh=mesh,
        scratch_types=[
            pltpu.VMEM((16, _L), jnp.float32),
            pltpu.VMEM((_C,), jnp.float32),
            pltpu.VMEM((_C,), jnp.float32),
            pltpu.VMEM((_C,), jnp.float32),
            pltpu.VMEM((_C,), jnp.float32),
            pltpu.SemaphoreType.DMA,
            pltpu.SemaphoreType.DMA,
            pltpu.SemaphoreType.DMA,
            pltpu.SemaphoreType.DMA,
            pltpu.SemaphoreType.DMA,
        ],
    )
    def quantize(x_hbm, tab_hbm, o_hbm, tab_v, in0, in1, out0, out1,
                 sem_t, si0, si1, so0, so1):
        wid = lax.axis_index("s") * _NC + lax.axis_index("c")
        base = wid * per_w

        pltpu.async_copy(tab_hbm, tab_v, sem_t).wait()
        m = [tab_v[i] for i in range(7)]
        v = [tab_v[7 + i] for i in range(8)]

        ins = (in0, in1)
        outs = (out0, out1)
        sis = (si0, si1)
        sos = (so0, so1)

        def start_in(chunk, slot):
            pltpu.make_async_copy(
                x_hbm.at[pl.ds(base + chunk * _C, _C)], ins[slot], sis[slot]
            ).start()

        def wait_in(slot):
            pltpu.make_async_copy(
                x_hbm.at[pl.ds(base, _C)], ins[slot], sis[slot]
            ).wait()

        def start_out(chunk, slot):
            pltpu.make_async_copy(
                outs[slot], o_hbm.at[pl.ds(base + chunk * _C, _C)], sos[slot]
            ).start()

        def wait_out(slot):
            pltpu.make_async_copy(
                outs[slot], o_hbm.at[pl.ds(base, _C)], sos[slot]
            ).wait()

        def compute(slot):
            src = ins[slot]
            dst = outs[slot]

            def cbody(i, carry):
                b = i * (_L * _U)
                for u in range(_U):
                    xv = src[pl.ds(b + u * _L, _L)]
                    acc = v[0]
                    for j in range(7):
                        acc = jnp.where(xv > m[j], v[j + 1], acc)
                    dst[pl.ds(b + u * _L, _L)] = acc
                return carry

            lax.fori_loop(0, _C // (_L * _U), cbody, 0, unroll=False)

        start_in(0, 0)
        start_in(1, 1)

        def pair_body(g, carry):
            for slot in range(2):
                chunk = 2 * g + slot
                wait_in(slot)

                @pl.when(g > 0)
                def _():
                    wait_out(slot)

                compute(slot)

                @pl.when(chunk + 2 < nchunks)
                def _():
                    start_in(chunk + 2, slot)

                start_out(chunk, slot)
            return carry

        lax.fori_loop(0, nchunks // 2, pair_body, 0, unroll=False)
        wait_out(0)
        wait_out(1)

    return quantize


@jax.jit
def kernel(x, fp3_values):
    sv = jnp.sort(fp3_values.astype(jnp.float32))
    mids = (sv[:-1] + sv[1:]) * jnp.float32(0.5)
    col = jnp.concatenate([mids, sv, jnp.zeros((1,), jnp.float32)])
    tab = jnp.broadcast_to(col[:, None], (16, _L))

    n = x.size
    flat = x.reshape(n)
    out = _make_sc_quantize(n)(flat, tab)
    return out.reshape(x.shape)
